# Initial kernel scaffold; baseline (speedup 1.0000x reference)
import jax
import jax.numpy as jnp
from jax import lax
from jax.experimental import pallas as pl

_N = 10000
_NGRAPH = 100
_NPG = 100
_K = 4

_f32 = jnp.float32


def _lrelu(h):
    return jnp.where(h >= 0, h, 0.01 * h)


def _topk_sels(pos):
    n = _NPG
    ones_col = jnp.ones((n, 1), _f32)
    d2 = None
    for c in range(3):
        pc = pos[:, c:c + 1]
        pj = lax.dot_general(ones_col, pc, (((1,), (1,)), ((), ())),
                             preferred_element_type=_f32)
        dc = pc - pj
        dc = dc * dc
        d2 = dc if d2 is None else d2 + dc
    ii = lax.broadcasted_iota(jnp.int32, (n, n), 0)
    jj = lax.broadcasted_iota(jnp.int32, (n, n), 1)
    d2 = d2 + jnp.where(ii == jj, _f32(1e10), _f32(0.0))
    sels = []
    for _ in range(_K):
        m = jnp.min(d2, axis=1, keepdims=True)
        cand = jnp.where(d2 <= m, jj, n)
        idx = jnp.min(cand, axis=1, keepdims=True)
        sel = jj == idx
        sels.append(sel.astype(_f32))
        d2 = jnp.where(sel, _f32(jnp.inf), d2)
    return sels


def _edge_conv(feat, sels, w1d, w1j, b1, w2, b2):
    a_node = jnp.dot(feat, w1d, preferred_element_type=_f32)
    b_node = jnp.dot(feat, w1j, preferred_element_type=_f32)
    acc = None
    for sel in sels:
        bj = jnp.dot(sel, b_node, preferred_element_type=_f32)
        h = _lrelu(a_node + bj + b1)
        h = _lrelu(jnp.dot(h, w2, preferred_element_type=_f32) + b2)
        acc = h if acc is None else acc + h
    return acc


def _body(x_ref,
          c1w1, c1b1, c1w2, c1b2,
          c2w1, c2b1, c2w2, c2b2,
          v1w1d, v1w1j, v1b1, v1w2, v1b2,
          v2w1d, v2w1j, v2b1, v2w2, v2b2,
          v3w1d, v3w1j, v3b1, v3w2, v3b2,
          v4w1d, v4w1j, v4b1, v4w2, v4b2,
          n1wx, n1wa, n1wb, n1wc, n1wd, n1b,
          n2w, n2b,
          n3wa, n3wb, n3wc, n3wd, n3b,
          n4w, n4b,
          o_ref):
    x = x_ref[0]
    h = _lrelu(jnp.dot(x, c1w1[...], preferred_element_type=_f32) + c1b1[...])
    h = _lrelu(jnp.dot(h, c1w2[...], preferred_element_type=_f32) + c1b2[...])
    h = _lrelu(h)
    h = _lrelu(jnp.dot(h, c2w1[...], preferred_element_type=_f32) + c2b1[...])
    h = _lrelu(jnp.dot(h, c2w2[...], preferred_element_type=_f32) + c2b2[...])
    xm = x * (h > 0).astype(_f32)

    sels = _topk_sels(xm[:, 0:3])
    a = _edge_conv(xm, sels, v1w1d[...], v1w1j[...], v1b1[...], v1w2[...], v1b2[...])
    sels = _topk_sels(a[:, 0:3])
    b = _edge_conv(a, sels, v2w1d[...], v2w1j[...], v2b1[...], v2w2[...], v2b2[...])
    sels = _topk_sels(b[:, 0:3])
    c = _edge_conv(b, sels, v3w1d[...], v3w1j[...], v3b1[...], v3w2[...], v3b2[...])
    sels = _topk_sels(c[:, 0:3])
    d = _edge_conv(c, sels, v4w1d[...], v4w1j[...], v4b1[...], v4w2[...], v4b2[...])

    h = (jnp.dot(xm, n1wx[...], preferred_element_type=_f32)
         + jnp.dot(a, n1wa[...], preferred_element_type=_f32)
         + jnp.dot(b, n1wb[...], preferred_element_type=_f32)
         + jnp.dot(c, n1wc[...], preferred_element_type=_f32)
         + jnp.dot(d, n1wd[...], preferred_element_type=_f32)
         + n1b[...])
    h = _lrelu(h)
    h = jnp.dot(h, n2w[...], preferred_element_type=_f32) + n2b[...]

    ga = jnp.max(h, axis=0, keepdims=True)
    gb = jnp.min(h, axis=0, keepdims=True)
    gc = jnp.sum(h, axis=0, keepdims=True)
    gd = gc / _f32(_NPG)
    t = (jnp.dot(_lrelu(ga), n3wa[...], preferred_element_type=_f32)
         + jnp.dot(_lrelu(gb), n3wb[...], preferred_element_type=_f32)
         + jnp.dot(_lrelu(gc), n3wc[...], preferred_element_type=_f32)
         + jnp.dot(_lrelu(gd), n3wd[...], preferred_element_type=_f32)
         + n3b[...])
    t = _lrelu(t)
    out = jnp.dot(t, n4w[...], preferred_element_type=_f32) + n4b[...]
    lane = lax.broadcasted_iota(jnp.int32, (1, 3), 1)
    out = jnp.where(lane < 2, jnp.tanh(out), out)
    o_ref[0] = out


def kernel(x, edge_index, batch, params):
    del edge_index, batch
    p = params

    def full(arr):
        return pl.BlockSpec(arr.shape, lambda *_: (0,) * arr.ndim)

    def b2(name):
        return p[name].reshape(1, -1)

    weights = []
    for pre in ('clean1', 'clean2'):
        weights += [p[pre + '_W1'], b2(pre + '_b1'), p[pre + '_W2'], b2(pre + '_b2')]
    for pre in ('conv1', 'conv2', 'conv3', 'conv4'):
        w1 = p[pre + '_W1']
        f = w1.shape[0] // 2
        weights += [w1[:f] - w1[f:], w1[f:], b2(pre + '_b1'),
                    p[pre + '_W2'], b2(pre + '_b2')]
    w = p['nn1_W']
    weights += [w[0:5], w[5:197], w[197:389], w[389:581], w[581:773], b2('nn1_b')]
    weights += [p['nn2_W'], b2('nn2_b')]
    w = p['nn3_W']
    weights += [w[0:192], w[192:384], w[384:576], w[576:768], b2('nn3_b')]
    weights += [p['nn4_W'], b2('nn4_b')]

    x3 = x.reshape(_NGRAPH, _NPG, 5)
    out = pl.pallas_call(
        _body,
        grid=(_NGRAPH,),
        in_specs=[pl.BlockSpec((1, _NPG, 5), lambda i: (i, 0, 0))]
                 + [full(wt) for wt in weights],
        out_specs=pl.BlockSpec((1, 1, 3), lambda i: (i, 0, 0)),
        out_shape=jax.ShapeDtypeStruct((_NGRAPH, 1, 3), _f32),
    )(x3, *weights)
    return out.reshape(_NGRAPH, 3)


# speedup vs baseline: 1.6520x; 1.6520x over previous
import jax
import jax.numpy as jnp
from jax import lax
from jax.experimental import pallas as pl

_N = 10000
_NGRAPH = 100
_NPG = 100
_K = 4

_f32 = jnp.float32
_bf16 = jnp.bfloat16
_EXACT = lax.Precision.HIGHEST


def _lrelu(h):
    return jnp.where(h >= 0, h, 0.01 * h)


def _bdot(a, w_bf16):
    return jnp.dot(a.astype(_bf16), w_bf16, preferred_element_type=_f32)


def _topk_sels(pos):
    n = _NPG
    ones_col = jnp.ones((n, 1), _f32)
    d2 = None
    for c in range(3):
        pc = pos[:, c:c + 1]
        pj = lax.dot_general(ones_col, pc, (((1,), (1,)), ((), ())),
                             preferred_element_type=_f32, precision=_EXACT)
        dc = pc - pj
        dc = dc * dc
        d2 = dc if d2 is None else d2 + dc
    ii = lax.broadcasted_iota(jnp.int32, (n, n), 0)
    jj = lax.broadcasted_iota(jnp.int32, (n, n), 1)
    d2 = d2 + jnp.where(ii == jj, _f32(1e10), _f32(0.0))
    sels = []
    for _ in range(_K):
        m = jnp.min(d2, axis=1, keepdims=True)
        cand = jnp.where(d2 <= m, jj, n)
        idx = jnp.min(cand, axis=1, keepdims=True)
        sel = jj == idx
        sels.append(sel.astype(_f32))
        d2 = jnp.where(sel, _f32(jnp.inf), d2)
    return sels


def _edge_conv(feat, sels, w1, b1, w2, b2):
    acc = None
    for sel in sels:
        xj = jnp.dot(sel, feat, preferred_element_type=_f32, precision=_EXACT)
        e = jnp.concatenate([feat, xj - feat], axis=1)
        h = _lrelu(_bdot(e, w1) + b1)
        h = _lrelu(_bdot(h, w2) + b2)
        acc = h if acc is None else acc + h
    return acc


def _body(x_ref,
          c1w1, c1b1, c1w2, c1b2,
          c2w1, c2b1, c2w2, c2b2,
          v1w1, v1b1, v1w2, v1b2,
          v2w1, v2b1, v2w2, v2b2,
          v3w1, v3b1, v3w2, v3b2,
          v4w1, v4b1, v4w2, v4b2,
          n1wx, n1wa, n1wb, n1wc, n1wd, n1b,
          n2w, n2b,
          n3wa, n3wb, n3wc, n3wd, n3b,
          n4w, n4b,
          o_ref):
    x = x_ref[0]
    h = _lrelu(_bdot(x, c1w1[...]) + c1b1[...])
    h = _lrelu(_bdot(h, c1w2[...]) + c1b2[...])
    h = _lrelu(h)
    h = _lrelu(_bdot(h, c2w1[...]) + c2b1[...])
    h = _lrelu(_bdot(h, c2w2[...]) + c2b2[...])
    xm = x * (h > 0).astype(_f32)

    sels = _topk_sels(xm[:, 0:3])
    a = _edge_conv(xm, sels, v1w1[...], v1b1[...], v1w2[...], v1b2[...])
    sels = _topk_sels(a[:, 0:3])
    b = _edge_conv(a, sels, v2w1[...], v2b1[...], v2w2[...], v2b2[...])
    sels = _topk_sels(b[:, 0:3])
    c = _edge_conv(b, sels, v3w1[...], v3b1[...], v3w2[...], v3b2[...])
    sels = _topk_sels(c[:, 0:3])
    d = _edge_conv(c, sels, v4w1[...], v4b1[...], v4w2[...], v4b2[...])

    h = (_bdot(xm, n1wx[...]) + _bdot(a, n1wa[...]) + _bdot(b, n1wb[...])
         + _bdot(c, n1wc[...]) + _bdot(d, n1wd[...]) + n1b[...])
    h = _lrelu(h)
    h = _bdot(h, n2w[...]) + n2b[...]

    ga = jnp.max(h, axis=0, keepdims=True)
    gb = jnp.min(h, axis=0, keepdims=True)
    gc = jnp.sum(h, axis=0, keepdims=True)
    gd = gc / _f32(_NPG)
    t = (_bdot(_lrelu(ga), n3wa[...]) + _bdot(_lrelu(gb), n3wb[...])
         + _bdot(_lrelu(gc), n3wc[...]) + _bdot(_lrelu(gd), n3wd[...])
         + n3b[...])
    t = _lrelu(t)
    out = _bdot(t, n4w[...]) + n4b[...]
    lane = lax.broadcasted_iota(jnp.int32, (1, 3), 1)
    out = jnp.where(lane < 2, jnp.tanh(out), out)
    o_ref[0] = out


def kernel(x, edge_index, batch, params):
    del edge_index, batch
    p = params

    def full(arr):
        return pl.BlockSpec(arr.shape, lambda *_: (0,) * arr.ndim)

    def b2(name):
        return p[name].reshape(1, -1)

    def wb(arr):
        return arr.astype(_bf16)

    weights = []
    for pre in ('clean1', 'clean2'):
        weights += [wb(p[pre + '_W1']), b2(pre + '_b1'),
                    wb(p[pre + '_W2']), b2(pre + '_b2')]
    for pre in ('conv1', 'conv2', 'conv3', 'conv4'):
        weights += [wb(p[pre + '_W1']), b2(pre + '_b1'),
                    wb(p[pre + '_W2']), b2(pre + '_b2')]
    w = p['nn1_W']
    weights += [wb(w[0:5]), wb(w[5:197]), wb(w[197:389]), wb(w[389:581]),
                wb(w[581:773]), b2('nn1_b')]
    weights += [wb(p['nn2_W']), b2('nn2_b')]
    w = p['nn3_W']
    weights += [wb(w[0:192]), wb(w[192:384]), wb(w[384:576]), wb(w[576:768]),
                b2('nn3_b')]
    weights += [wb(p['nn4_W']), b2('nn4_b')]

    x3 = x.reshape(_NGRAPH, _NPG, 5)
    out = pl.pallas_call(
        _body,
        grid=(_NGRAPH,),
        in_specs=[pl.BlockSpec((1, _NPG, 5), lambda i: (i, 0, 0))]
                 + [full(wt) for wt in weights],
        out_specs=pl.BlockSpec((1, 1, 3), lambda i: (i, 0, 0)),
        out_shape=jax.ShapeDtypeStruct((_NGRAPH, 1, 3), _f32),
    )(x3, *weights)
    return out.reshape(_NGRAPH, 3)


# speedup vs baseline: 2.1912x; 1.3264x over previous
import jax
import jax.numpy as jnp
from jax import lax
from jax.experimental import pallas as pl

_N = 10000
_NGRAPH = 100
_NPG = 100
_K = 4

_f32 = jnp.float32
_bf16 = jnp.bfloat16
_EXACT = lax.Precision.HIGHEST


def _lrelu(h):
    return jnp.where(h >= 0, h, 0.01 * h)


def _bdot(a, w_bf16):
    return jnp.dot(a.astype(_bf16), w_bf16, preferred_element_type=_f32)


def _topk_sels(pos):
    n = _NPG
    pos_t = jnp.transpose(pos)
    d2 = None
    for c in range(3):
        dc = pos[:, c:c + 1] - pos_t[c:c + 1, :]
        dc = dc * dc
        d2 = dc if d2 is None else d2 + dc
    ii = lax.broadcasted_iota(jnp.int32, (n, n), 0)
    jj = lax.broadcasted_iota(jnp.int32, (n, n), 1)
    d2 = d2 + jnp.where(ii == jj, _f32(1e10), _f32(0.0))
    sels = []
    for _ in range(_K):
        m = jnp.min(d2, axis=1, keepdims=True)
        cand = jnp.where(d2 <= m, jj, n)
        idx = jnp.min(cand, axis=1, keepdims=True)
        sel = jj == idx
        sels.append(sel.astype(_bf16))
        d2 = jnp.where(sel, _f32(jnp.inf), d2)
    return sels


def _edge_conv(feat, sels, w1, b1, w2, b2):
    hi = feat.astype(_bf16)
    r1 = feat - hi.astype(_f32)
    md = r1.astype(_bf16)
    lo = (r1 - md.astype(_f32)).astype(_bf16)
    acc = None
    for sel in sels:
        xj = (jnp.dot(sel, hi, preferred_element_type=_f32)
              + jnp.dot(sel, md, preferred_element_type=_f32)
              + jnp.dot(sel, lo, preferred_element_type=_f32))
        e = jnp.concatenate([feat, xj - feat], axis=1)
        h = _lrelu(_bdot(e, w1) + b1)
        h = _lrelu(_bdot(h, w2) + b2)
        acc = h if acc is None else acc + h
    return acc


def _body(x_ref,
          c1w1, c1b1, c1w2, c1b2,
          c2w1, c2b1, c2w2, c2b2,
          v1w1, v1b1, v1w2, v1b2,
          v2w1, v2b1, v2w2, v2b2,
          v3w1, v3b1, v3w2, v3b2,
          v4w1, v4b1, v4w2, v4b2,
          n1wx, n1wa, n1wb, n1wc, n1wd, n1b,
          n2w, n2b,
          n3wa, n3wb, n3wc, n3wd, n3b,
          n4w, n4b,
          o_ref):
    x = x_ref[0]
    h = _lrelu(_bdot(x, c1w1[...]) + c1b1[...])
    h = _lrelu(_bdot(h, c1w2[...]) + c1b2[...])
    h = _lrelu(h)
    h = _lrelu(_bdot(h, c2w1[...]) + c2b1[...])
    h = _lrelu(_bdot(h, c2w2[...]) + c2b2[...])
    xm = x * (h > 0).astype(_f32)

    sels = _topk_sels(xm[:, 0:3])
    a = _edge_conv(xm, sels, v1w1[...], v1b1[...], v1w2[...], v1b2[...])
    sels = _topk_sels(a[:, 0:3])
    b = _edge_conv(a, sels, v2w1[...], v2b1[...], v2w2[...], v2b2[...])
    sels = _topk_sels(b[:, 0:3])
    c = _edge_conv(b, sels, v3w1[...], v3b1[...], v3w2[...], v3b2[...])
    sels = _topk_sels(c[:, 0:3])
    d = _edge_conv(c, sels, v4w1[...], v4b1[...], v4w2[...], v4b2[...])

    h = (_bdot(xm, n1wx[...]) + _bdot(a, n1wa[...]) + _bdot(b, n1wb[...])
         + _bdot(c, n1wc[...]) + _bdot(d, n1wd[...]) + n1b[...])
    h = _lrelu(h)
    h = _bdot(h, n2w[...]) + n2b[...]

    ga = jnp.max(h, axis=0, keepdims=True)
    gb = jnp.min(h, axis=0, keepdims=True)
    gc = jnp.sum(h, axis=0, keepdims=True)
    gd = gc / _f32(_NPG)
    t = (_bdot(_lrelu(ga), n3wa[...]) + _bdot(_lrelu(gb), n3wb[...])
         + _bdot(_lrelu(gc), n3wc[...]) + _bdot(_lrelu(gd), n3wd[...])
         + n3b[...])
    t = _lrelu(t)
    out = _bdot(t, n4w[...]) + n4b[...]
    lane = lax.broadcasted_iota(jnp.int32, (1, 3), 1)
    out = jnp.where(lane < 2, jnp.tanh(out), out)
    o_ref[0] = out


def kernel(x, edge_index, batch, params):
    del edge_index, batch
    p = params

    def full(arr):
        return pl.BlockSpec(arr.shape, lambda *_: (0,) * arr.ndim)

    def b2(name):
        return p[name].reshape(1, -1)

    def wb(arr):
        return arr.astype(_bf16)

    weights = []
    for pre in ('clean1', 'clean2'):
        weights += [wb(p[pre + '_W1']), b2(pre + '_b1'),
                    wb(p[pre + '_W2']), b2(pre + '_b2')]
    for pre in ('conv1', 'conv2', 'conv3', 'conv4'):
        weights += [wb(p[pre + '_W1']), b2(pre + '_b1'),
                    wb(p[pre + '_W2']), b2(pre + '_b2')]
    w = p['nn1_W']
    weights += [wb(w[0:5]), wb(w[5:197]), wb(w[197:389]), wb(w[389:581]),
                wb(w[581:773]), b2('nn1_b')]
    weights += [wb(p['nn2_W']), b2('nn2_b')]
    w = p['nn3_W']
    weights += [wb(w[0:192]), wb(w[192:384]), wb(w[384:576]), wb(w[576:768]),
                b2('nn3_b')]
    weights += [wb(p['nn4_W']), b2('nn4_b')]

    x3 = x.reshape(_NGRAPH, _NPG, 5)
    out = pl.pallas_call(
        _body,
        grid=(_NGRAPH,),
        in_specs=[pl.BlockSpec((1, _NPG, 5), lambda i: (i, 0, 0))]
                 + [full(wt) for wt in weights],
        out_specs=pl.BlockSpec((1, 1, 3), lambda i: (i, 0, 0)),
        out_shape=jax.ShapeDtypeStruct((_NGRAPH, 1, 3), _f32),
    )(x3, *weights)
    return out.reshape(_NGRAPH, 3)


# speedup vs baseline: 2.2010x; 1.0045x over previous
import jax
import jax.numpy as jnp
from jax import lax
from jax.experimental import pallas as pl

_N = 10000
_NGRAPH = 100
_NPG = 100
_K = 4
_BG = 2

_f32 = jnp.float32
_bf16 = jnp.bfloat16
_EXACT = lax.Precision.HIGHEST


def _lrelu(h):
    return jnp.where(h >= 0, h, 0.01 * h)


def _bdot(a, w_bf16):
    return jnp.dot(a.astype(_bf16), w_bf16, preferred_element_type=_f32)


def _topk_sels(pos):
    n = _NPG
    pos_t = jnp.transpose(pos)
    d2 = None
    for c in range(3):
        dc = pos[:, c:c + 1] - pos_t[c:c + 1, :]
        dc = dc * dc
        d2 = dc if d2 is None else d2 + dc
    ii = lax.broadcasted_iota(jnp.int32, (n, n), 0)
    jj = lax.broadcasted_iota(jnp.int32, (n, n), 1)
    d2 = d2 + jnp.where(ii == jj, _f32(1e10), _f32(0.0))
    sels = []
    for _ in range(_K):
        m = jnp.min(d2, axis=1, keepdims=True)
        cand = jnp.where(d2 <= m, jj, n)
        idx = jnp.min(cand, axis=1, keepdims=True)
        sel = jj == idx
        sels.append(sel.astype(_bf16))
        d2 = jnp.where(sel, _f32(jnp.inf), d2)
    return sels


def _edge_conv(feat, sels, w1, b1, w2, b2):
    hi = feat.astype(_bf16)
    r1 = feat - hi.astype(_f32)
    md = r1.astype(_bf16)
    lo = (r1 - md.astype(_f32)).astype(_bf16)
    acc = None
    for sel in sels:
        xj = (jnp.dot(sel, hi, preferred_element_type=_f32)
              + jnp.dot(sel, md, preferred_element_type=_f32)
              + jnp.dot(sel, lo, preferred_element_type=_f32))
        e = jnp.concatenate([feat, xj - feat], axis=1)
        h = _lrelu(_bdot(e, w1) + b1)
        h = _lrelu(_bdot(h, w2) + b2)
        acc = h if acc is None else acc + h
    return acc


def _body(x_ref,
          c1w1, c1b1, c1w2, c1b2,
          c2w1, c2b1, c2w2, c2b2,
          v1w1, v1b1, v1w2, v1b2,
          v2w1, v2b1, v2w2, v2b2,
          v3w1, v3b1, v3w2, v3b2,
          v4w1, v4b1, v4w2, v4b2,
          n1wx, n1wa, n1wb, n1wc, n1wd, n1b,
          n2w, n2b,
          n3wa, n3wb, n3wc, n3wd, n3b,
          n4w, n4b,
          o_ref):
    for g in range(_BG):
        o_ref[g] = _graph(
            x_ref[g],
            c1w1, c1b1, c1w2, c1b2, c2w1, c2b1, c2w2, c2b2,
            v1w1, v1b1, v1w2, v1b2, v2w1, v2b1, v2w2, v2b2,
            v3w1, v3b1, v3w2, v3b2, v4w1, v4b1, v4w2, v4b2,
            n1wx, n1wa, n1wb, n1wc, n1wd, n1b, n2w, n2b,
            n3wa, n3wb, n3wc, n3wd, n3b, n4w, n4b)


def _graph(x,
           c1w1, c1b1, c1w2, c1b2,
           c2w1, c2b1, c2w2, c2b2,
           v1w1, v1b1, v1w2, v1b2,
           v2w1, v2b1, v2w2, v2b2,
           v3w1, v3b1, v3w2, v3b2,
           v4w1, v4b1, v4w2, v4b2,
           n1wx, n1wa, n1wb, n1wc, n1wd, n1b,
           n2w, n2b,
           n3wa, n3wb, n3wc, n3wd, n3b,
           n4w, n4b):
    h = _lrelu(_bdot(x, c1w1[...]) + c1b1[...])
    h = _lrelu(_bdot(h, c1w2[...]) + c1b2[...])
    h = _lrelu(h)
    h = _lrelu(_bdot(h, c2w1[...]) + c2b1[...])
    h = _lrelu(_bdot(h, c2w2[...]) + c2b2[...])
    xm = x * (h > 0).astype(_f32)

    sels = _topk_sels(xm[:, 0:3])
    a = _edge_conv(xm, sels, v1w1[...], v1b1[...], v1w2[...], v1b2[...])
    sels = _topk_sels(a[:, 0:3])
    b = _edge_conv(a, sels, v2w1[...], v2b1[...], v2w2[...], v2b2[...])
    sels = _topk_sels(b[:, 0:3])
    c = _edge_conv(b, sels, v3w1[...], v3b1[...], v3w2[...], v3b2[...])
    sels = _topk_sels(c[:, 0:3])
    d = _edge_conv(c, sels, v4w1[...], v4b1[...], v4w2[...], v4b2[...])

    h = (_bdot(xm, n1wx[...]) + _bdot(a, n1wa[...]) + _bdot(b, n1wb[...])
         + _bdot(c, n1wc[...]) + _bdot(d, n1wd[...]) + n1b[...])
    h = _lrelu(h)
    h = _bdot(h, n2w[...]) + n2b[...]

    ga = jnp.max(h, axis=0, keepdims=True)
    gb = jnp.min(h, axis=0, keepdims=True)
    gc = jnp.sum(h, axis=0, keepdims=True)
    gd = gc / _f32(_NPG)
    t = (_bdot(_lrelu(ga), n3wa[...]) + _bdot(_lrelu(gb), n3wb[...])
         + _bdot(_lrelu(gc), n3wc[...]) + _bdot(_lrelu(gd), n3wd[...])
         + n3b[...])
    t = _lrelu(t)
    out = _bdot(t, n4w[...]) + n4b[...]
    lane = lax.broadcasted_iota(jnp.int32, (1, 3), 1)
    return jnp.where(lane < 2, jnp.tanh(out), out)


def kernel(x, edge_index, batch, params):
    del edge_index, batch
    p = params

    def full(arr):
        return pl.BlockSpec(arr.shape, lambda *_: (0,) * arr.ndim)

    def b2(name):
        return p[name].reshape(1, -1)

    def wb(arr):
        return arr.astype(_bf16)

    weights = []
    for pre in ('clean1', 'clean2'):
        weights += [wb(p[pre + '_W1']), b2(pre + '_b1'),
                    wb(p[pre + '_W2']), b2(pre + '_b2')]
    for pre in ('conv1', 'conv2', 'conv3', 'conv4'):
        weights += [wb(p[pre + '_W1']), b2(pre + '_b1'),
                    wb(p[pre + '_W2']), b2(pre + '_b2')]
    w = p['nn1_W']
    weights += [wb(w[0:5]), wb(w[5:197]), wb(w[197:389]), wb(w[389:581]),
                wb(w[581:773]), b2('nn1_b')]
    weights += [wb(p['nn2_W']), b2('nn2_b')]
    w = p['nn3_W']
    weights += [wb(w[0:192]), wb(w[192:384]), wb(w[384:576]), wb(w[576:768]),
                b2('nn3_b')]
    weights += [wb(p['nn4_W']), b2('nn4_b')]

    x3 = x.reshape(_NGRAPH, _NPG, 5)
    out = pl.pallas_call(
        _body,
        grid=(_NGRAPH // _BG,),
        in_specs=[pl.BlockSpec((_BG, _NPG, 5), lambda i: (i, 0, 0))]
                 + [full(wt) for wt in weights],
        out_specs=pl.BlockSpec((_BG, 1, 3), lambda i: (i, 0, 0)),
        out_shape=jax.ShapeDtypeStruct((_NGRAPH, 1, 3), _f32),
    )(x3, *weights)
    return out.reshape(_NGRAPH, 3)
